# Initial kernel scaffold; baseline (speedup 1.0000x reference)
import jax
import jax.numpy as jnp
from jax.experimental import pallas as pl


def kernel(node_features, receivers):
    raise NotImplementedError("write your pallas kernel here")


# speedup vs baseline: 3.6470x; 3.6470x over previous
import functools

import jax
import jax.numpy as jnp
from jax import lax
from jax.experimental import pallas as pl
from jax.experimental.pallas import tpu as pltpu
from jax.experimental.pallas import tpu_sc as plsc

_NUM_NODES = 160000
_DIM = 256
_NSEG = 32
_LANES = 16

_NC = 2
_NS = 16
_COLS = _DIM // _NC
_ROWS = _NUM_NODES // _NS
_CHUNK = 200
_NCHUNK = _ROWS // _CHUNK
_NBUF = 2
_CVEC = _COLS // _LANES
_BSEARCH_STEPS = 14


def _segment_sum_sc(node_features, receivers):
    mesh = plsc.VectorSubcoreMesh(core_axis_name="c", subcore_axis_name="s")

    @functools.partial(
        pl.kernel,
        mesh=mesh,
        out_type=jax.ShapeDtypeStruct((_NSEG, _DIM), jnp.float32),
        compiler_params=pltpu.CompilerParams(needs_layout_passes=False),
        scratch_types=[
            pltpu.VMEM((_ROWS,), jnp.int32),
            pltpu.VMEM((_CHUNK, _COLS), jnp.float32),
            pltpu.VMEM((_CHUNK, _COLS), jnp.float32),
            pltpu.VMEM((_NSEG, _COLS), jnp.float32),
            pltpu.VMEM((_NSEG,), jnp.int32),
            pltpu.VMEM_SHARED((_NSEG, _COLS), jnp.float32),
            pltpu.SemaphoreType.DMA,
            pltpu.SemaphoreType.DMA,
        ],
    )
    def k(nf_hbm, recv_hbm, out_hbm, recv_v, buf0, buf1, acc, idx_v, shared,
          sem0, sem1):
        cid = lax.axis_index("c")
        sid = lax.axis_index("s")
        row0 = sid * _ROWS
        col0 = cid * _COLS
        bufs = (buf0, buf1)
        sems = (sem0, sem1)

        zeros = jnp.zeros((_LANES,), jnp.float32)
        for s in range(_NSEG):
            for j in range(_CVEC):
                acc[s, pl.ds(j * _LANES, _LANES)] = zeros

        lane = lax.broadcasted_iota(jnp.int32, (_LANES,), 0)
        for j in range(_NSEG // _LANES):
            idx_v[pl.ds(j * _LANES, _LANES)] = lane + j * _LANES

        @pl.when(sid == 0)
        def _():
            pltpu.sync_copy(acc, shared)

        plsc.subcore_barrier()

        pltpu.sync_copy(recv_hbm.at[pl.ds(row0, _ROWS)], recv_v)

        bounds = []
        for h in range(_NSEG // _LANES):
            seg = lane + h * _LANES
            lo = jnp.zeros((_LANES,), jnp.int32)
            hi = jnp.full((_LANES,), _ROWS, jnp.int32)
            for _ in range(_BSEARCH_STEPS):
                active = lo < hi
                mid = (lo + hi) >> 1
                midc = jnp.minimum(mid, _ROWS - 1)
                vals = plsc.load_gather(recv_v, [midc])
                go = vals < seg
                lo = jnp.where(active & go, mid + 1, lo)
                hi = jnp.where(active & (~go), mid, hi)
            bounds.append(lo)

        b = []
        for s in range(_NSEG):
            vec = bounds[s // _LANES]
            b.append(jnp.max(jnp.where(lane == (s % _LANES), vec, 0)))
        b.append(jnp.int32(_ROWS))

        def chunk_copy(k_idx, bi):
            src = nf_hbm.at[pl.ds(row0 + k_idx * _CHUNK, _CHUNK),
                            pl.ds(col0, _COLS)]
            return pltpu.make_async_copy(src, bufs[bi], sems[bi])

        for bi in range(_NBUF):
            chunk_copy(bi, bi).start()

        def outer(g, _):
            for bi in range(_NBUF):
                k_idx = g * _NBUF + bi
                chunk_copy(k_idx, bi).wait()
                base = k_idx * _CHUNK
                buf = bufs[bi]
                for s in range(_NSEG):
                    lo_s = jnp.clip(b[s] - base, 0, _CHUNK)
                    hi_s = jnp.clip(b[s + 1] - base, 0, _CHUNK)

                    @pl.when(hi_s > lo_s)
                    def _(s=s, lo_s=lo_s, hi_s=hi_s, buf=buf):
                        def row_body(i, carry):
                            return tuple(
                                carry[j] + buf[i, pl.ds(j * _LANES, _LANES)]
                                for j in range(_CVEC))

                        carry = lax.fori_loop(
                            lo_s, hi_s, row_body,
                            tuple(jnp.zeros((_LANES,), jnp.float32)
                                  for _ in range(_CVEC)))
                        for j in range(_CVEC):
                            sl = pl.ds(j * _LANES, _LANES)
                            acc[s, sl] = acc[s, sl] + carry[j]

                @pl.when(k_idx + _NBUF < _NCHUNK)
                def _(k_idx=k_idx, bi=bi):
                    chunk_copy(k_idx + _NBUF, bi).start()
            return None

        lax.fori_loop(0, _NCHUNK // _NBUF, outer, None)

        pltpu.sync_copy(acc, shared.at[idx_v], add=True)
        plsc.subcore_barrier()

        @pl.when(sid == 0)
        def _():
            pltpu.sync_copy(shared,
                            out_hbm.at[pl.ds(0, _NSEG), pl.ds(col0, _COLS)])

    return k(node_features, receivers)


@jax.jit
def kernel(node_features, receivers):
    if receivers.ndim == 2:
        receivers = receivers[:, 0]
    return _segment_sum_sc(node_features, receivers)
